# baseline (speedup 1.0000x reference)
import functools

import jax
import jax.numpy as jnp
from jax import lax
from jax.experimental import pallas as pl
from jax.experimental.pallas import tpu as pltpu
from jax.experimental.pallas import tpu_sc as plsc

N = 10000
NP = 10240
E = 160000
EP = 163840
D = 64
B = 1024
NW = 32
EPW = EP // NW
CH = 128
NCH = EPW // CH
RPT = NP // 16

_f32 = jnp.float32

def _mesh():
    return plsc.VectorSubcoreMesh(core_axis_name="c", subcore_axis_name="s")


def _sc_gather(table, idx3):
    @functools.partial(
        pl.kernel,
        mesh=_mesh(),
        out_type=jax.ShapeDtypeStruct((EP, 2 * D), _f32),
        scratch_types=[
            pltpu.VMEM((NCH, CH), jnp.int32),
            pltpu.VMEM((CH, 2 * D), _f32),
            pltpu.SemaphoreType.DMA,
        ],
    )
    def k(table_hbm, idx_hbm, out_hbm, idx_v, rows_v, sem):
        c = lax.axis_index("c")
        s = lax.axis_index("s")
        wid = s * 2 + c
        pltpu.sync_copy(idx_hbm.at[wid], idx_v)
        base = wid * EPW

        def body(ch, carry):
            pltpu.async_copy(table_hbm.at[idx_v.at[ch]], rows_v, sem).wait()
            pltpu.sync_copy(rows_v, out_hbm.at[pl.ds(base + ch * CH, CH)])
            return carry

        lax.fori_loop(0, NCH, body, 0)

    return k(table, idx3)


def _sc_scatter_add(values, idx3, zeros_tile):
    width = 2 * D

    @functools.partial(
        pl.kernel,
        mesh=_mesh(),
        out_type=jax.ShapeDtypeStruct((2, NP, width), _f32),
        scratch_types=[
            pltpu.VMEM((CH,), jnp.int32),
            pltpu.VMEM((CH, width), _f32),
            pltpu.VMEM_SHARED((NP, width), _f32),
        ],
    )
    def k(val_hbm, idx_hbm, z_hbm, out_hbm, idx_v, buf_v, agg_sh):
        c = lax.axis_index("c")
        s = lax.axis_index("s")
        wid = s * 2 + c
        base = wid * EPW
        row0 = s * RPT
        pltpu.sync_copy(z_hbm, agg_sh.at[pl.ds(row0, RPT)])
        plsc.subcore_barrier()

        def body(ch, carry):
            pltpu.sync_copy(idx_hbm.at[wid, ch], idx_v)
            pltpu.sync_copy(val_hbm.at[pl.ds(base + ch * CH, CH)], buf_v)
            pltpu.sync_copy(buf_v, agg_sh.at[idx_v], add=True)
            return carry

        lax.fori_loop(0, NCH, body, 0)
        plsc.subcore_barrier()
        pltpu.sync_copy(agg_sh.at[pl.ds(row0, RPT)], out_hbm.at[c, pl.ds(row0, RPT)])

    return k(values, idx3, zeros_tile)


def _lin_relu_body(x_ref, w_ref, b_ref, o_ref):
    o_ref[...] = jnp.maximum(
        jnp.dot(x_ref[...], w_ref[...], preferred_element_type=_f32) + b_ref[...],
        0.0)


def _wide_lin_relu_body(x_ref, w_ref, b_ref, o_ref):
    res = jnp.maximum(
        jnp.dot(x_ref[...], w_ref[...], preferred_element_type=_f32) + b_ref[...],
        0.0)
    o_ref[...] = jnp.concatenate([res, jnp.zeros_like(res)], axis=1)


def _pre(xp, w, b):
    return pl.pallas_call(
        _wide_lin_relu_body,
        out_shape=jax.ShapeDtypeStruct((NP, 2 * D), _f32),
    )(xp, w, b)


def _mlp1(eap, w, b):
    blk = 4096
    return pl.pallas_call(
        _lin_relu_body,
        grid=(EP // blk,),
        in_specs=[
            pl.BlockSpec((blk, 16), lambda i: (i, 0)),
            pl.BlockSpec((16, 128), lambda i: (0, 0)),
            pl.BlockSpec((1, 128), lambda i: (0, 0)),
        ],
        out_specs=pl.BlockSpec((blk, 128), lambda i: (i, 0)),
        out_shape=jax.ShapeDtypeStruct((EP, 128), _f32),
    )(eap, w, b)


_EB = 256


def _msg_body(xj_ref, h1_ref, w2_ref, b2_ref, o_ref):
    w = jnp.dot(h1_ref[...], w2_ref[...], preferred_element_type=_f32) + b2_ref[...]
    xj = xj_ref[...][:, 0:D]
    acc = xj[:, 0:1] * w[:, 0:D]
    for i in range(1, D):
        acc = acc + xj[:, i:i + 1] * w[:, D * i:D * (i + 1)]
    ones = jnp.ones((acc.shape[0], 1), _f32)
    zeros = jnp.zeros((acc.shape[0], D - 1), _f32)
    o_ref[...] = jnp.concatenate([acc, ones, zeros], axis=1)


def _msg(xj, h1, w2, b2):
    return pl.pallas_call(
        _msg_body,
        grid=(EP // _EB,),
        in_specs=[
            pl.BlockSpec((_EB, 2 * D), lambda i: (i, 0)),
            pl.BlockSpec((_EB, 128), lambda i: (i, 0)),
            pl.BlockSpec((128, D * D), lambda i: (0, 0)),
            pl.BlockSpec((1, D * D), lambda i: (0, 0)),
        ],
        out_specs=pl.BlockSpec((_EB, 2 * D), lambda i: (i, 0)),
        out_shape=jax.ShapeDtypeStruct((EP, 2 * D), _f32),
    )(xj, h1, w2, b2)


def _update_body(a0, a1, h_ref, cw, cb, wir, wiz, win, whr, whz, whn,
                 br, bz, bi_n, bh_n, o_ref):
    asum = a0[...] + a1[...]
    cnt = jnp.maximum(asum[:, D:D + 1], 1.0)
    mean = asum[:, 0:D] / cnt
    h = h_ref[...][:, 0:D]
    m = jnp.maximum(
        mean + jnp.dot(h, cw[...], preferred_element_type=_f32) + cb[...], 0.0)
    r = jax.nn.sigmoid(jnp.dot(m, wir[...], preferred_element_type=_f32)
                       + jnp.dot(h, whr[...], preferred_element_type=_f32)
                       + br[...])
    z = jax.nn.sigmoid(jnp.dot(m, wiz[...], preferred_element_type=_f32)
                       + jnp.dot(h, whz[...], preferred_element_type=_f32)
                       + bz[...])
    hn = jnp.dot(h, whn[...], preferred_element_type=_f32) + bh_n[...]
    n = jnp.tanh(jnp.dot(m, win[...], preferred_element_type=_f32)
                 + bi_n[...] + r * hn)
    res = (1.0 - z) * n + z * h
    o_ref[...] = jnp.concatenate([res, jnp.zeros_like(res)], axis=1)


def _update(a0, a1, h, cw, cb, giw, ghw, gb):
    blk = 1024
    mat = pl.BlockSpec((D, D), lambda i: (0, 0))
    row = pl.BlockSpec((1, D), lambda i: (0, 0))
    wide = pl.BlockSpec((blk, 2 * D), lambda i: (i, 0))
    return pl.pallas_call(
        _update_body,
        grid=(NP // blk,),
        in_specs=[
            wide, wide, wide, mat, row,
            mat, mat, mat, mat, mat, mat,
            row, row, row, row,
        ],
        out_specs=wide,
        out_shape=jax.ShapeDtypeStruct((NP, 2 * D), _f32),
    )(a0, a1, h, cw, cb, *giw, *ghw, *gb)


def _attn_body(out_ref, bt_ref, q_ref, a_ref, ao_ref):
    bt = bt_ref[...]
    cols = lax.broadcasted_iota(jnp.int32, (bt.shape[0], B), 1)
    sm = (bt == cols).astype(_f32)
    qn = jnp.dot(sm, q_ref[...], preferred_element_type=_f32)
    o = out_ref[...][:, 0:D]
    e = jnp.sum(o * qn, axis=1, keepdims=True)
    a = jnp.exp(e)
    a_ref[...] = a
    ao_ref[...] = a * o


def _attn(out, bt2, q):
    blk = 1024
    return pl.pallas_call(
        _attn_body,
        grid=(NP // blk,),
        in_specs=[
            pl.BlockSpec((blk, 2 * D), lambda i: (i, 0)),
            pl.BlockSpec((blk, 1), lambda i: (i, 0)),
            pl.BlockSpec((B, D), lambda i: (0, 0)),
        ],
        out_specs=[
            pl.BlockSpec((blk, 1), lambda i: (i, 0)),
            pl.BlockSpec((blk, D), lambda i: (i, 0)),
        ],
        out_shape=[
            jax.ShapeDtypeStruct((NP, 1), _f32),
            jax.ShapeDtypeStruct((NP, D), _f32),
        ],
    )(out, bt2, q)


def _seg_body(a_ref, ao_ref, bt_ref, rv_ref, as_ref):
    i = pl.program_id(0)

    @pl.when(i == 0)
    def _():
        rv_ref[...] = jnp.zeros_like(rv_ref)
        as_ref[...] = jnp.zeros_like(as_ref)

    bt = bt_ref[0]
    rows = lax.broadcasted_iota(jnp.int32, (B, bt.shape[1]), 0)
    sm = (rows == bt).astype(_f32)
    rv_ref[...] += jnp.dot(sm, ao_ref[...], preferred_element_type=_f32)
    as_ref[...] += jnp.dot(sm, a_ref[...], preferred_element_type=_f32)

    @pl.when(i == pl.num_programs(0) - 1)
    def _():
        rv_ref[...] = rv_ref[...] / (as_ref[...] + 1e-16)


def _seg(a, ao, bt3):
    blk = 1024
    rv, _ = pl.pallas_call(
        _seg_body,
        grid=(NP // blk,),
        in_specs=[
            pl.BlockSpec((blk, 1), lambda i: (i, 0)),
            pl.BlockSpec((blk, D), lambda i: (i, 0)),
            pl.BlockSpec((1, 1, blk), lambda i: (i, 0, 0)),
        ],
        out_specs=[
            pl.BlockSpec((B, D), lambda i: (0, 0)),
            pl.BlockSpec((B, 1), lambda i: (0, 0)),
        ],
        out_shape=[
            jax.ShapeDtypeStruct((B, D), _f32),
            jax.ShapeDtypeStruct((B, 1), _f32),
        ],
        compiler_params=pltpu.CompilerParams(
            dimension_semantics=("arbitrary",)),
    )(a, ao, bt3)
    return rv


def _lstm_body(qs_ref, h_ref, c_ref, wii, wif, wig, wio, whi, whf, whg, who,
               bi, bf, bg, bo, ho_ref, co_ref):
    qs = qs_ref[...]
    h = h_ref[...]

    def gate(wi, wh, bb):
        return (jnp.dot(qs, wi[...], preferred_element_type=_f32)
                + jnp.dot(h, wh[...], preferred_element_type=_f32) + bb[...])

    gi = jax.nn.sigmoid(gate(wii, whi, bi))
    gf = jax.nn.sigmoid(gate(wif, whf, bf))
    gg = jnp.tanh(gate(wig, whg, bg))
    go = jax.nn.sigmoid(gate(wio, who, bo))
    cc = gf * c_ref[...] + gi * gg
    co_ref[...] = cc
    ho_ref[...] = go * jnp.tanh(cc)


def _lstm(qs, h, c, wi, wh, bs):
    return pl.pallas_call(
        _lstm_body,
        out_shape=[
            jax.ShapeDtypeStruct((B, D), _f32),
            jax.ShapeDtypeStruct((B, D), _f32),
        ],
    )(qs, h, c, *wi, *wh, *bs)


def _head_body(qs_ref, w1, b1, w2r, b2, y_ref):
    t = jnp.maximum(
        jnp.dot(qs_ref[...], w1[...], preferred_element_type=_f32) + b1[...],
        0.0)
    y_ref[...] = jnp.sum(t * w2r[...], axis=1, keepdims=True) + b2[...]


def _head(qs, w1, b1, w2r, b2):
    return pl.pallas_call(
        _head_body,
        out_shape=jax.ShapeDtypeStruct((B, 1), _f32),
    )(qs, w1, b1, w2r, b2)


def kernel(x, edge_index, edge_attr, batch, lin0_W, lin0_b, nn1_W, nn1_b,
           nn2_W, nn2_b, conv_root, conv_bias, gru_Wih, gru_Whh, gru_bih,
           gru_bhh, lstm_Wih, lstm_Whh, lstm_bih, lstm_bhh, lin1_W, lin1_b,
           lin2_W, lin2_b):
    i32 = jnp.int32
    src = edge_index[0]
    dst = edge_index[1]
    src3 = jnp.concatenate([src, jnp.zeros((EP - E,), i32)]).reshape(NW, NCH, CH)
    dst3 = jnp.concatenate([dst, jnp.full((EP - E,), N, i32)]).reshape(NW, NCH, CH)
    eap = jnp.pad(edge_attr, ((0, EP - E), (0, 0)))
    xp = jnp.pad(x, ((0, NP - N), (0, 11)))
    batp = jnp.pad(batch, (0, NP - N), constant_values=B)
    bt2 = batp.reshape(NP, 1)
    bt3 = batp.reshape(NP // 1024, 1, 1024)

    lin0_Wp = jnp.pad(lin0_W.T, ((0, 11), (0, 0)))
    out = _pre(xp, lin0_Wp, lin0_b[None, :])
    h1 = _mlp1(eap, nn1_W.T, nn1_b[None, :])

    wir = gru_Wih[0:D].T
    wiz = gru_Wih[D:2 * D].T
    win = gru_Wih[2 * D:].T
    whr = gru_Whh[0:D].T
    whz = gru_Whh[D:2 * D].T
    whn = gru_Whh[2 * D:].T
    br = (gru_bih[0:D] + gru_bhh[0:D])[None, :]
    bz = (gru_bih[D:2 * D] + gru_bhh[D:2 * D])[None, :]
    bi_n = gru_bih[2 * D:][None, :]
    bh_n = gru_bhh[2 * D:][None, :]

    zm = jnp.zeros((RPT, 2 * D), _f32)
    w2 = nn2_W.T
    b2 = nn2_b[None, :]
    cw = conv_root.T
    cb = conv_bias[None, :]
    for _ in range(3):
        xj = _sc_gather(out, src3)
        msg = _msg(xj, h1, w2, b2)
        aggs = _sc_scatter_add(msg, dst3, zm)
        out = _update(aggs[0], aggs[1], out, cw, cb,
                      (wir, wiz, win), (whr, whz, whn), (br, bz, bi_n, bh_n))

    wi_g = tuple(lstm_Wih[k * D:(k + 1) * D].T for k in range(4))
    wh_g = tuple(lstm_Whh[k * D:(k + 1) * D].T for k in range(4))
    b_g = tuple((lstm_bih[k * D:(k + 1) * D] + lstm_bhh[k * D:(k + 1) * D])[None, :]
                for k in range(4))
    hB = jnp.zeros((B, D), _f32)
    cB = jnp.zeros((B, D), _f32)
    qs = jnp.zeros((B, 2 * D), _f32)
    for _ in range(3):
        hB, cB = _lstm(qs, hB, cB, wi_g, wh_g, b_g)
        a, ao = _attn(out, bt2, hB)
        rvec = _seg(a, ao, bt3)
        qs = jnp.concatenate([hB, rvec], axis=1)

    y = _head(qs, lin1_W.T, lin1_b[None, :], lin2_W[0][None, :], lin2_b[None, :])
    return y.reshape(-1)
